# baseline (speedup 1.0000x reference)
import functools

import jax
import jax.numpy as jnp
from jax import lax
from jax.experimental import pallas as pl
from jax.experimental.pallas import tpu as pltpu
from jax.experimental.pallas import tpu_sc as plsc

VOCAB = 100000
D_EMB = 128
BATCH = 1024
MAXN = 1.0

_NC = 2
_NS = 16
_NW = _NC * _NS
_BPW = BATCH // _NW


def _sc_gather(x, table):
    mesh = plsc.VectorSubcoreMesh(core_axis_name="c", subcore_axis_name="s")

    @functools.partial(
        pl.kernel,
        mesh=mesh,
        out_type=jax.ShapeDtypeStruct((BATCH, D_EMB), jnp.float32),
        scratch_types=[
            pltpu.VMEM((_BPW,), jnp.int32),
            pltpu.VMEM((_BPW, D_EMB), jnp.float32),
            pltpu.SemaphoreType.DMA,
        ],
    )
    def gk(idx_hbm, table_hbm, out_hbm, idx_v, rows_v, sem):
        wid = lax.axis_index("s") * _NC + lax.axis_index("c")
        base = wid * _BPW
        pltpu.sync_copy(idx_hbm.at[pl.ds(base, _BPW)], idx_v)
        pltpu.async_copy(table_hbm.at[idx_v], rows_v, sem).wait()
        pltpu.sync_copy(rows_v, out_hbm.at[pl.ds(base, _BPW)])

    return gk(x, table)


_VB = 1024


def _mm_body(e_ref, w_ref, b_ref, o_ref):
    e = e_ref[...]
    n = jnp.sqrt(jnp.sum(e * e, axis=1, keepdims=True))
    scale = jnp.where(n > MAXN, MAXN / jnp.maximum(n, 1e-7), 1.0)
    e = e * scale
    acc = lax.dot_general(
        e, w_ref[...], (((1,), (1,)), ((), ())),
        preferred_element_type=jnp.float32,
    )
    o_ref[...] = acc + b_ref[...]


def _tc_project(e, W, b2):
    grid = pl.cdiv(VOCAB, _VB)
    return pl.pallas_call(
        _mm_body,
        grid=(grid,),
        in_specs=[
            pl.BlockSpec((BATCH, D_EMB), lambda i: (0, 0)),
            pl.BlockSpec((_VB, D_EMB), lambda i: (i, 0)),
            pl.BlockSpec((1, _VB), lambda i: (0, i)),
        ],
        out_specs=pl.BlockSpec((BATCH, _VB), lambda i: (0, i)),
        out_shape=jax.ShapeDtypeStruct((BATCH, VOCAB), jnp.float32),
    )(e, W, b2)


def kernel(x, table, W, b):
    e = _sc_gather(x.astype(jnp.int32), table)
    return _tc_project(e, W, b.reshape(1, VOCAB))


# speedup vs baseline: 1.0271x; 1.0271x over previous
import functools

import jax
import jax.numpy as jnp
from jax import lax
from jax.experimental import pallas as pl
from jax.experimental.pallas import tpu as pltpu
from jax.experimental.pallas import tpu_sc as plsc

VOCAB = 100000
D_EMB = 128
BATCH = 1024
MAXN = 1.0

_NC = 2
_NS = 16
_NW = _NC * _NS
_BPW = BATCH // _NW


def _sc_gather(x, table):
    mesh = plsc.VectorSubcoreMesh(core_axis_name="c", subcore_axis_name="s")

    @functools.partial(
        pl.kernel,
        mesh=mesh,
        out_type=jax.ShapeDtypeStruct((BATCH, D_EMB), jnp.float32),
        scratch_types=[
            pltpu.VMEM((_BPW,), jnp.int32),
            pltpu.VMEM((_BPW, D_EMB), jnp.float32),
            pltpu.SemaphoreType.DMA,
        ],
    )
    def gk(idx_hbm, table_hbm, out_hbm, idx_v, rows_v, sem):
        wid = lax.axis_index("s") * _NC + lax.axis_index("c")
        base = wid * _BPW
        pltpu.sync_copy(idx_hbm.at[pl.ds(base, _BPW)], idx_v)
        pltpu.async_copy(table_hbm.at[idx_v], rows_v, sem).wait()
        pltpu.sync_copy(rows_v, out_hbm.at[pl.ds(base, _BPW)])

    return gk(x, table)


_VB = 1024
_NBUF = 4
_NFULL = VOCAB // _VB
_TAIL = VOCAB - _NFULL * _VB
_GRID = _NFULL + 1


def _mm_body(e_ref, w_ref, b_ref, o_ref, es_ref, obuf_ref, tail_ref, sem_ref):
    i = pl.program_id(0)

    @pl.when(i == 0)
    def _():
        e = e_ref[...]
        n = jnp.sqrt(jnp.sum(e * e, axis=1, keepdims=True))
        scale = jnp.where(n > MAXN, MAXN / jnp.maximum(n, 1e-7), 1.0)
        es_ref[...] = (e * scale).astype(jnp.bfloat16)

    slot = lax.rem(i, _NBUF)

    @pl.when(i >= _NBUF)
    def _():
        pltpu.make_async_copy(
            obuf_ref.at[slot],
            o_ref.at[:, pl.ds(0, _VB)],
            sem_ref.at[slot],
        ).wait()

    acc = lax.dot_general(
        es_ref[...], w_ref[...].astype(jnp.bfloat16),
        (((1,), (1,)), ((), ())),
        preferred_element_type=jnp.float32,
    )
    acc = acc + b_ref[...]

    @pl.when(i < _NFULL)
    def _():
        obuf_ref[slot] = acc
        pltpu.make_async_copy(
            obuf_ref.at[slot],
            o_ref.at[:, pl.ds(i * _VB, _VB)],
            sem_ref.at[slot],
        ).start()

    @pl.when(i == _NFULL)
    def _():
        tail_ref[...] = acc[:, :_TAIL]
        pltpu.make_async_copy(
            tail_ref,
            o_ref.at[:, pl.ds(_NFULL * _VB, _TAIL)],
            sem_ref.at[slot],
        ).start()
        tail_slot = _NFULL % _NBUF
        for s in range(_NBUF):
            if s == tail_slot:
                pltpu.make_async_copy(
                    tail_ref,
                    o_ref.at[:, pl.ds(_NFULL * _VB, _TAIL)],
                    sem_ref.at[s],
                ).wait()
            else:
                pltpu.make_async_copy(
                    obuf_ref.at[s],
                    o_ref.at[:, pl.ds(0, _VB)],
                    sem_ref.at[s],
                ).wait()


def _tc_project(e, W, b2):
    return pl.pallas_call(
        _mm_body,
        grid=(_GRID,),
        in_specs=[
            pl.BlockSpec((BATCH, D_EMB), lambda i: (0, 0)),
            pl.BlockSpec((_VB, D_EMB), lambda i: (i, 0)),
            pl.BlockSpec((1, _VB), lambda i: (0, i)),
        ],
        out_specs=pl.BlockSpec(memory_space=pl.ANY),
        out_shape=jax.ShapeDtypeStruct((BATCH, VOCAB), jnp.float32),
        scratch_shapes=[
            pltpu.VMEM((BATCH, D_EMB), jnp.bfloat16),
            pltpu.VMEM((_NBUF, BATCH, _VB), jnp.float32),
            pltpu.VMEM((BATCH, _TAIL), jnp.float32),
            pltpu.SemaphoreType.DMA((_NBUF,)),
        ],
    )(e, W, b2)


def kernel(x, table, W, b):
    e = _sc_gather(x.astype(jnp.int32), table)
    return _tc_project(e, W, b.reshape(1, VOCAB))


# speedup vs baseline: 2.2759x; 2.2158x over previous
import functools

import jax
import jax.numpy as jnp
from jax import lax
from jax.experimental import pallas as pl
from jax.experimental.pallas import tpu as pltpu
from jax.experimental.pallas import tpu_sc as plsc

VOCAB = 100000
D_EMB = 128
BATCH = 1024
MAXN = 1.0

_NC = 2
_NS = 16
_NW = _NC * _NS
_BPW = BATCH // _NW


def _sc_gather(x, table):
    mesh = plsc.VectorSubcoreMesh(core_axis_name="c", subcore_axis_name="s")

    @functools.partial(
        pl.kernel,
        mesh=mesh,
        out_type=jax.ShapeDtypeStruct((BATCH, D_EMB), jnp.float32),
        scratch_types=[
            pltpu.VMEM((_BPW,), jnp.int32),
            pltpu.VMEM((_BPW, D_EMB), jnp.float32),
            pltpu.SemaphoreType.DMA,
        ],
    )
    def gk(idx_hbm, table_hbm, out_hbm, idx_v, rows_v, sem):
        wid = lax.axis_index("s") * _NC + lax.axis_index("c")
        base = wid * _BPW
        pltpu.sync_copy(idx_hbm.at[pl.ds(base, _BPW)], idx_v)
        pltpu.async_copy(table_hbm.at[idx_v], rows_v, sem).wait()
        pltpu.sync_copy(rows_v, out_hbm.at[pl.ds(base, _BPW)])

    return gk(x, table)


_VB = 1024


def _mm_body(e_ref, w_ref, b_ref, o_ref, es_ref):
    @pl.when(pl.program_id(0) == 0)
    def _():
        e = e_ref[...]
        n = jnp.sqrt(jnp.sum(e * e, axis=1, keepdims=True))
        scale = jnp.where(n > MAXN, MAXN / jnp.maximum(n, 1e-7), 1.0)
        es_ref[...] = (e * scale).astype(jnp.bfloat16)

    acc = lax.dot_general(
        w_ref[...].astype(jnp.bfloat16), es_ref[...],
        (((1,), (1,)), ((), ())),
        preferred_element_type=jnp.float32,
    )
    o_ref[...] = acc + b_ref[...]


def _tc_project_t(e, W, bc):
    grid = pl.cdiv(VOCAB, _VB)
    return pl.pallas_call(
        _mm_body,
        grid=(grid,),
        in_specs=[
            pl.BlockSpec((BATCH, D_EMB), lambda i: (0, 0)),
            pl.BlockSpec((_VB, D_EMB), lambda i: (i, 0)),
            pl.BlockSpec((_VB, 1), lambda i: (i, 0)),
        ],
        out_specs=pl.BlockSpec((_VB, BATCH), lambda i: (i, 0)),
        out_shape=jax.ShapeDtypeStruct((VOCAB, BATCH), jnp.float32),
        scratch_shapes=[pltpu.VMEM((BATCH, D_EMB), jnp.bfloat16)],
    )(e, W, bc)


def kernel(x, table, W, b):
    e = _sc_gather(x.astype(jnp.int32), table)
    logits_t = _tc_project_t(e, W, b.reshape(VOCAB, 1))
    return logits_t.T


# speedup vs baseline: 2.8615x; 1.2573x over previous
import functools

import jax
import jax.numpy as jnp
from jax import lax
from jax.experimental import pallas as pl
from jax.experimental.pallas import tpu as pltpu
from jax.experimental.pallas import tpu_sc as plsc

VOCAB = 100000
D_EMB = 128
BATCH = 1024
MAXN = 1.0

_NC = 2
_NS = 16
_NW = _NC * _NS
_BPW = BATCH // _NW


def _sc_gather(x, table):
    mesh = plsc.VectorSubcoreMesh(core_axis_name="c", subcore_axis_name="s")

    @functools.partial(
        pl.kernel,
        mesh=mesh,
        out_type=jax.ShapeDtypeStruct((BATCH, D_EMB), jnp.float32),
        scratch_types=[
            pltpu.VMEM((_BPW,), jnp.int32),
            pltpu.VMEM((_BPW, D_EMB), jnp.float32),
            pltpu.SemaphoreType.DMA,
        ],
    )
    def gk(idx_hbm, table_hbm, out_hbm, idx_v, rows_v, sem):
        wid = lax.axis_index("s") * _NC + lax.axis_index("c")
        base = wid * _BPW
        pltpu.sync_copy(idx_hbm.at[pl.ds(base, _BPW)], idx_v)
        pltpu.async_copy(table_hbm.at[idx_v], rows_v, sem).wait()
        pltpu.sync_copy(rows_v, out_hbm.at[pl.ds(base, _BPW)])

    return gk(x, table)


_VB = 1024


def _mm_body(e_ref, w_ref, b_ref, o_ref, es_ref):
    @pl.when(pl.program_id(0) == 0)
    def _():
        e = e_ref[...]
        n = jnp.sqrt(jnp.sum(e * e, axis=1, keepdims=True))
        scale = jnp.where(n > MAXN, MAXN / jnp.maximum(n, 1e-7), 1.0)
        es_ref[...] = (e * scale).astype(jnp.bfloat16)

    acc = lax.dot_general(
        w_ref[...].astype(jnp.bfloat16), es_ref[...],
        (((1,), (1,)), ((), ())),
        preferred_element_type=jnp.float32,
    )
    o_ref[...] = acc + b_ref[...].T


def _tc_project_t(e, W, bc):
    grid = pl.cdiv(VOCAB, _VB)
    return pl.pallas_call(
        _mm_body,
        grid=(grid,),
        in_specs=[
            pl.BlockSpec((BATCH, D_EMB), lambda i: (0, 0)),
            pl.BlockSpec((_VB, D_EMB), lambda i: (i, 0)),
            pl.BlockSpec((1, _VB), lambda i: (0, i)),
        ],
        out_specs=pl.BlockSpec((_VB, BATCH), lambda i: (i, 0)),
        out_shape=jax.ShapeDtypeStruct((VOCAB, BATCH), jnp.float32),
        scratch_shapes=[pltpu.VMEM((BATCH, D_EMB), jnp.bfloat16)],
    )(e, W, bc)


def kernel(x, table, W, b):
    e = _sc_gather(x.astype(jnp.int32), table)
    logits_t = _tc_project_t(e, W, b.reshape(1, VOCAB))
    return logits_t.T


# speedup vs baseline: 3.3365x; 1.1660x over previous
import functools

import jax
import jax.numpy as jnp
from jax import lax
from jax.experimental import pallas as pl
from jax.experimental.pallas import tpu as pltpu
from jax.experimental.pallas import tpu_sc as plsc

VOCAB = 100000
D_EMB = 128
BATCH = 1024
MAXN = 1.0

_NC = 2
_NS = 16
_NW = _NC * _NS
_BPW = BATCH // _NW


def _sc_gather(x, table):
    mesh = plsc.VectorSubcoreMesh(core_axis_name="c", subcore_axis_name="s")

    @functools.partial(
        pl.kernel,
        mesh=mesh,
        out_type=jax.ShapeDtypeStruct((BATCH, D_EMB), jnp.float32),
        scratch_types=[
            pltpu.VMEM((_BPW,), jnp.int32),
            pltpu.VMEM((_BPW, D_EMB), jnp.float32),
            pltpu.SemaphoreType.DMA,
        ],
    )
    def gk(idx_hbm, table_hbm, out_hbm, idx_v, rows_v, sem):
        wid = lax.axis_index("s") * _NC + lax.axis_index("c")
        base = wid * _BPW
        pltpu.sync_copy(idx_hbm.at[pl.ds(base, _BPW)], idx_v)
        pltpu.async_copy(table_hbm.at[idx_v], rows_v, sem).wait()
        pltpu.sync_copy(rows_v, out_hbm.at[pl.ds(base, _BPW)])

    return gk(x, table)


_VB = 4096


def _mm_body(e_ref, w_ref, b_ref, o_ref, es_ref):
    @pl.when(pl.program_id(0) == 0)
    def _():
        e = e_ref[...]
        n = jnp.sqrt(jnp.sum(e * e, axis=1, keepdims=True))
        scale = jnp.where(n > MAXN, MAXN / jnp.maximum(n, 1e-7), 1.0)
        es_ref[...] = (e * scale).astype(jnp.bfloat16)

    acc = lax.dot_general(
        w_ref[...].astype(jnp.bfloat16), es_ref[...],
        (((1,), (1,)), ((), ())),
        preferred_element_type=jnp.float32,
    )
    o_ref[...] = acc + b_ref[...].T


def _tc_project_t(e, W, bc):
    grid = pl.cdiv(VOCAB, _VB)
    return pl.pallas_call(
        _mm_body,
        grid=(grid,),
        in_specs=[
            pl.BlockSpec((BATCH, D_EMB), lambda i: (0, 0)),
            pl.BlockSpec((_VB, D_EMB), lambda i: (i, 0)),
            pl.BlockSpec((1, _VB), lambda i: (0, i)),
        ],
        out_specs=pl.BlockSpec((_VB, BATCH), lambda i: (i, 0)),
        out_shape=jax.ShapeDtypeStruct((VOCAB, BATCH), jnp.float32),
        scratch_shapes=[pltpu.VMEM((BATCH, D_EMB), jnp.bfloat16)],
    )(e, W, bc)


def kernel(x, table, W, b):
    e = _sc_gather(x.astype(jnp.int32), table)
    logits_t = _tc_project_t(e, W, b.reshape(1, VOCAB))
    return logits_t.T


# speedup vs baseline: 3.3520x; 1.0047x over previous
import functools

import jax
import jax.numpy as jnp
from jax import lax
from jax.experimental import pallas as pl
from jax.experimental.pallas import tpu as pltpu
from jax.experimental.pallas import tpu_sc as plsc

VOCAB = 100000
D_EMB = 128
BATCH = 1024
MAXN = 1.0

_NC = 2
_NS = 16
_NW = _NC * _NS
_BPW = BATCH // _NW


def _sc_gather(x, table):
    mesh = plsc.VectorSubcoreMesh(core_axis_name="c", subcore_axis_name="s")

    @functools.partial(
        pl.kernel,
        mesh=mesh,
        out_type=jax.ShapeDtypeStruct((BATCH, D_EMB), jnp.float32),
        scratch_types=[
            pltpu.VMEM((_BPW,), jnp.int32),
            pltpu.VMEM((_BPW, D_EMB), jnp.float32),
            pltpu.SemaphoreType.DMA,
        ],
    )
    def gk(idx_hbm, table_hbm, out_hbm, idx_v, rows_v, sem):
        wid = lax.axis_index("s") * _NC + lax.axis_index("c")
        base = wid * _BPW
        pltpu.sync_copy(idx_hbm.at[pl.ds(base, _BPW)], idx_v)
        pltpu.async_copy(table_hbm.at[idx_v], rows_v, sem).wait()
        pltpu.sync_copy(rows_v, out_hbm.at[pl.ds(base, _BPW)])

    return gk(x, table)


_VB = 5632


def _mm_body(e_ref, w_ref, b_ref, o_ref, es_ref):
    @pl.when(pl.program_id(0) == 0)
    def _():
        e = e_ref[...]
        n = jnp.sqrt(jnp.sum(e * e, axis=1, keepdims=True))
        scale = jnp.where(n > MAXN, MAXN / jnp.maximum(n, 1e-7), 1.0)
        es_ref[...] = (e * scale).astype(jnp.bfloat16)

    acc = lax.dot_general(
        w_ref[...].astype(jnp.bfloat16), es_ref[...],
        (((1,), (1,)), ((), ())),
        preferred_element_type=jnp.float32,
    )
    o_ref[...] = acc + b_ref[...].T


def _tc_project_t(e, W, bc):
    grid = pl.cdiv(VOCAB, _VB)
    return pl.pallas_call(
        _mm_body,
        grid=(grid,),
        in_specs=[
            pl.BlockSpec((BATCH, D_EMB), lambda i: (0, 0)),
            pl.BlockSpec((_VB, D_EMB), lambda i: (i, 0)),
            pl.BlockSpec((1, _VB), lambda i: (0, i)),
        ],
        out_specs=pl.BlockSpec((_VB, BATCH), lambda i: (i, 0)),
        out_shape=jax.ShapeDtypeStruct((VOCAB, BATCH), jnp.float32),
        scratch_shapes=[pltpu.VMEM((BATCH, D_EMB), jnp.bfloat16)],
    )(e, W, bc)


def kernel(x, table, W, b):
    e = _sc_gather(x.astype(jnp.int32), table)
    logits_t = _tc_project_t(e, W, b.reshape(1, VOCAB))
    return logits_t.T
